# Initial kernel scaffold; baseline (speedup 1.0000x reference)
import jax
import jax.numpy as jnp
from jax.experimental import pallas as pl


def kernel(nodes, edges, senders, receivers, pn_w1, pn_b1, pn_w2, pn_b2, pe_w1, pe_b1, pe_w2, pe_b2, em_w1, em_b1, em_w2, em_b2, em_w3, em_b3, nm_w1, nm_b1, nm_w2, nm_b2, nm_w3, nm_b3, gm_w1, gm_b1, gm_w2, gm_b2, gm_w3, gm_b3):
    raise NotImplementedError("write your pallas kernel here")


# speedup vs baseline: 3.6658x; 3.6658x over previous
import functools

import jax
import jax.numpy as jnp
from jax import lax
from jax.experimental import pallas as pl
from jax.experimental.pallas import tpu as pltpu
from jax.experimental.pallas import tpu_sc as plsc

F = 16
_NC = 2
_NS = 16
_NW = _NC * _NS
_SELU_ALPHA = 1.6732632423543772
_SELU_SCALE = 1.0507009873554805


def _selu(x):
    return _SELU_SCALE * jnp.where(x > 0, x, _SELU_ALPHA * (jnp.exp(x) - 1.0))


def _pick_block(total, target):
    b = 8
    for d in range(8, target + 1, 8):
        if total % d == 0:
            b = d
    return b


def _node_enc_body(x_ref, w1, b1, w2, b2, out_ref):
    h = _selu(jnp.dot(x_ref[...], w1[...], preferred_element_type=jnp.float32) + b1[...])
    out_ref[...] = jnp.dot(h, w2[...], preferred_element_type=jnp.float32) + b2[...]


def _edge_body(le_ref, gr_ref, gs_ref, w1, b1, w2, b2, w3, b3, out_ref):
    x = jnp.concatenate([le_ref[...], gr_ref[...], gs_ref[...]], axis=1)
    h = _selu(jnp.dot(x, w1[...], preferred_element_type=jnp.float32) + b1[...])
    h = _selu(jnp.dot(h, w2[...], preferred_element_type=jnp.float32) + b2[...])
    out_ref[...] = jnp.dot(h, w3[...], preferred_element_type=jnp.float32) + b3[...]


def _edge1_body(ed_ref, gr_ref, gs_ref, ew1, eb1, ew2, eb2,
                w1, b1, w2, b2, w3, b3, out_ref):
    le = jnp.dot(_selu(jnp.dot(ed_ref[...], ew1[...], preferred_element_type=jnp.float32) + eb1[...]),
                 ew2[...], preferred_element_type=jnp.float32) + eb2[...]
    x = jnp.concatenate([le, gr_ref[...], gs_ref[...]], axis=1)
    h = _selu(jnp.dot(x, w1[...], preferred_element_type=jnp.float32) + b1[...])
    h = _selu(jnp.dot(h, w2[...], preferred_element_type=jnp.float32) + b2[...])
    out_ref[...] = jnp.dot(h, w3[...], preferred_element_type=jnp.float32) + b3[...]


def _node_body(agg0_ref, agg1_ref, ln_ref, w1, b1, w2, b2, w3, b3, out_ref):
    agg = agg0_ref[...] + agg1_ref[...]
    x = jnp.concatenate([agg, ln_ref[...]], axis=1)
    h = _selu(jnp.dot(x, w1[...], preferred_element_type=jnp.float32) + b1[...])
    h = _selu(jnp.dot(h, w2[...], preferred_element_type=jnp.float32) + b2[...])
    out_ref[...] = jnp.dot(h, w3[...], preferred_element_type=jnp.float32) + b3[...]


def _node_final_body(agg0_ref, agg1_ref, ln_ref, w1, b1, w2, b2, w3, b3,
                     gw1, gb1, gw2, gb2, gw3, gb3,
                     out_ref, glob_ref, acc_e, acc_n):
    i = pl.program_id(0)
    agg = agg0_ref[...] + agg1_ref[...]
    x = jnp.concatenate([agg, ln_ref[...]], axis=1)
    h = _selu(jnp.dot(x, w1[...], preferred_element_type=jnp.float32) + b1[...])
    h = _selu(jnp.dot(h, w2[...], preferred_element_type=jnp.float32) + b2[...])
    out = jnp.dot(h, w3[...], preferred_element_type=jnp.float32) + b3[...]
    out_ref[...] = out
    pe = jnp.sum(agg, axis=0, keepdims=True)
    pn = jnp.sum(out, axis=0, keepdims=True)

    @pl.when(i == 0)
    def _():
        acc_e[...] = pe
        acc_n[...] = pn

    @pl.when(i != 0)
    def _():
        acc_e[...] = acc_e[...] + pe
        acc_n[...] = acc_n[...] + pn

    @pl.when(i == pl.num_programs(0) - 1)
    def _():
        xg = jnp.concatenate([acc_e[...], acc_n[...]], axis=1)
        g = _selu(jnp.dot(xg, gw1[...], preferred_element_type=jnp.float32) + gb1[...])
        g = _selu(jnp.dot(g, gw2[...], preferred_element_type=jnp.float32) + gb2[...])
        glob_ref[...] = jnp.dot(g, gw3[...], preferred_element_type=jnp.float32) + gb3[...]


def _full_spec(shape):
    return pl.BlockSpec(shape, lambda i: (0,) * len(shape))


def _row_spec(b, w, off_blocks=0):
    if off_blocks:
        return pl.BlockSpec((b, w), lambda i: (i + off_blocks, 0))
    return pl.BlockSpec((b, w), lambda i: (i, 0))


def _make_gather(n, e, chunk):
    epw = e // _NW
    nch = epw // chunk
    mesh = plsc.VectorSubcoreMesh(core_axis_name="c", subcore_axis_name="s",
                                  num_cores=_NC, num_subcores=_NS)

    @functools.partial(
        pl.kernel,
        out_type=(jax.ShapeDtypeStruct((e, F), jnp.float32),
                  jax.ShapeDtypeStruct((e, F), jnp.float32)),
        mesh=mesh,
        scratch_types=[pltpu.VMEM((chunk,), jnp.int32),
                       pltpu.VMEM((chunk,), jnp.int32),
                       pltpu.VMEM((chunk, F), jnp.float32),
                       pltpu.VMEM((chunk, F), jnp.float32),
                       pltpu.SemaphoreType.DMA],
        compiler_params=pltpu.CompilerParams(use_tc_tiling_on_sc=False),
    )
    def gather(latn_hbm, snd_hbm, rcv_hbm, gs_hbm, gr_hbm,
               idx_s, idx_r, rows_s, rows_r, sem):
        wid = lax.axis_index("s") * _NC + lax.axis_index("c")
        base = wid * epw

        def body(j, carry):
            off = base + j * chunk
            pltpu.sync_copy(rcv_hbm.at[pl.ds(off, chunk)], idx_r)
            pltpu.sync_copy(snd_hbm.at[pl.ds(off, chunk)], idx_s)
            d1 = pltpu.async_copy(latn_hbm.at[idx_r], rows_r, sem)
            d2 = pltpu.async_copy(latn_hbm.at[idx_s], rows_s, sem)
            d1.wait()
            d2.wait()
            pltpu.sync_copy(rows_r, gr_hbm.at[pl.ds(off, chunk)])
            pltpu.sync_copy(rows_s, gs_hbm.at[pl.ds(off, chunk)])
            return carry

        lax.fori_loop(0, nch, body, 0)

    return gather


def _make_scatter(n, e, chunk):
    epw = e // _NW
    nch = epw // chunk
    tr = n // _NS
    mesh = plsc.VectorSubcoreMesh(core_axis_name="c", subcore_axis_name="s",
                                  num_cores=_NC, num_subcores=_NS)

    @functools.partial(
        pl.kernel,
        out_type=jax.ShapeDtypeStruct((_NC * n, F), jnp.float32),
        mesh=mesh,
        scratch_types=[pltpu.VMEM((chunk,), jnp.int32),
                       pltpu.VMEM((chunk, F), jnp.float32),
                       pltpu.VMEM_SHARED((n, F), jnp.float32),
                       pltpu.SemaphoreType.DMA],
        compiler_params=pltpu.CompilerParams(use_tc_tiling_on_sc=False),
    )
    def scatter(late_hbm, rcv_hbm, agg_hbm, idx_v, rows_v, table, sem):
        c = lax.axis_index("c")
        s = lax.axis_index("s")
        base = (s * _NC + c) * epw

        def zero(i, carry):
            rows_v[i] = jnp.zeros((F,), jnp.float32)
            return carry

        lax.fori_loop(0, chunk, zero, 0)
        row0 = s * tr
        done = 0
        while done + chunk <= tr:
            pltpu.sync_copy(rows_v, table.at[pl.ds(row0 + done, chunk)])
            done += chunk
        if done < tr:
            pltpu.sync_copy(rows_v.at[pl.ds(0, tr - done)],
                            table.at[pl.ds(row0 + done, tr - done)])
        plsc.subcore_barrier()

        def body(j, carry):
            off = base + j * chunk
            pltpu.sync_copy(rcv_hbm.at[pl.ds(off, chunk)], idx_v)
            pltpu.sync_copy(late_hbm.at[pl.ds(off, chunk)], rows_v)
            pltpu.sync_copy(rows_v, table.at[idx_v], add=True)
            return carry

        lax.fori_loop(0, nch, body, 0)
        plsc.subcore_barrier()
        pltpu.sync_copy(table.at[pl.ds(row0, tr)],
                        agg_hbm.at[pl.ds(c * n + row0, tr)])

    return scatter


def kernel(nodes, edges, senders, receivers, pn_w1, pn_b1, pn_w2, pn_b2,
           pe_w1, pe_b1, pe_w2, pe_b2, em_w1, em_b1, em_w2, em_b2, em_w3,
           em_b3, nm_w1, nm_b1, nm_w2, nm_b2, nm_w3, nm_b3, gm_w1, gm_b1,
           gm_w2, gm_b2, gm_w3, gm_b3):
    n, e = nodes.shape[0], edges.shape[0]
    f32 = jnp.float32
    senders = senders.astype(jnp.int32)
    receivers = receivers.astype(jnp.int32)

    def padc(w, width):
        return jnp.pad(w, ((0, 0), (0, width - w.shape[1])))

    def padb(b, width):
        return jnp.pad(b, (0, width - b.shape[0]))[None]

    pn_w2p, pn_b2p = padc(pn_w2, F), padb(pn_b2, F)
    pe_w2p, pe_b2p = padc(pe_w2, F), padb(pe_b2, F)
    ew1 = jnp.zeros((3 * F, 64), f32)
    ew1 = ew1.at[0:10].set(em_w1[0:10]).at[F:F + 10].set(em_w1[10:20])
    ew1 = ew1.at[2 * F:2 * F + 10].set(em_w1[20:30])
    eb1 = em_b1[None]
    eb2 = em_b2[None]
    ew3, eb3 = padc(em_w3, F), padb(em_b3, F)
    nw1 = jnp.zeros((2 * F, 64), f32)
    nw1 = nw1.at[0:10].set(nm_w1[0:10]).at[F:F + 10].set(nm_w1[10:20])
    nb1 = nm_b1[None]
    nb2 = nm_b2[None]
    nw3, nb3 = padc(nm_w3, F), padb(nm_b3, F)
    gw1 = jnp.zeros((2 * F, 64), f32)
    gw1 = gw1.at[0:10].set(gm_w1[0:10]).at[F:F + 10].set(gm_w1[10:20])
    gb1 = gm_b1[None]
    gb2 = gm_b2[None]
    gw3, gb3 = padc(gm_w3, F), padb(gm_b3, F)

    nb = _pick_block(n, 5000)
    ebk = _pick_block(e, 8000)
    n_grid = n // nb
    e_grid = e // ebk

    lat_n = pl.pallas_call(
        _node_enc_body,
        grid=(n_grid,),
        in_specs=[_row_spec(nb, 3), _full_spec((3, 10)), _full_spec((1, 10)),
                  _full_spec((10, F)), _full_spec((1, F))],
        out_specs=_row_spec(nb, F),
        out_shape=jax.ShapeDtypeStruct((n, F), f32),
    )(nodes, pn_w1, pn_b1[None], pn_w2p, pn_b2p)

    chunk = 2000
    gather_fn = _make_gather(n, e, chunk)
    scatter_fn = _make_scatter(n, e, chunk)

    edge_w = (ew1, eb1, em_w2, eb2, ew3, eb3)
    edge_w_specs = [_full_spec((3 * F, 64)), _full_spec((1, 64)),
                    _full_spec((64, 32)), _full_spec((1, 32)),
                    _full_spec((32, F)), _full_spec((1, F))]
    node_w = (nw1, nb1, nm_w2, nb2, nw3, nb3)
    node_w_specs = [_full_spec((2 * F, 64)), _full_spec((1, 64)),
                    _full_spec((64, 32)), _full_spec((1, 32)),
                    _full_spec((32, F)), _full_spec((1, F))]
    glob_w = (gw1, gb1, gm_w2, gb2, gw3, gb3)
    glob_w_specs = list(node_w_specs)

    lat_e = None
    glob = None
    for r in range(3):
        gs, gr = gather_fn(lat_n, senders, receivers)
        if r == 0:
            lat_e = pl.pallas_call(
                _edge1_body,
                grid=(e_grid,),
                in_specs=[_row_spec(ebk, 3), _row_spec(ebk, F), _row_spec(ebk, F),
                          _full_spec((3, 10)), _full_spec((1, 10)),
                          _full_spec((10, F)), _full_spec((1, F))] + edge_w_specs,
                out_specs=_row_spec(ebk, F),
                out_shape=jax.ShapeDtypeStruct((e, F), f32),
            )(edges, gr, gs, pe_w1, pe_b1[None], pe_w2p, pe_b2p, *edge_w)
        else:
            lat_e = pl.pallas_call(
                _edge_body,
                grid=(e_grid,),
                in_specs=[_row_spec(ebk, F), _row_spec(ebk, F), _row_spec(ebk, F)]
                + edge_w_specs,
                out_specs=_row_spec(ebk, F),
                out_shape=jax.ShapeDtypeStruct((e, F), f32),
            )(lat_e, gr, gs, *edge_w)
        agg2 = scatter_fn(lat_e, receivers)
        if r < 2:
            lat_n = pl.pallas_call(
                _node_body,
                grid=(n_grid,),
                in_specs=[_row_spec(nb, F), _row_spec(nb, F, n_grid),
                          _row_spec(nb, F)] + node_w_specs,
                out_specs=_row_spec(nb, F),
                out_shape=jax.ShapeDtypeStruct((n, F), f32),
            )(agg2, agg2, lat_n, *node_w)
        else:
            lat_n, glob = pl.pallas_call(
                _node_final_body,
                grid=(n_grid,),
                in_specs=[_row_spec(nb, F), _row_spec(nb, F, n_grid),
                          _row_spec(nb, F)] + node_w_specs + glob_w_specs,
                out_specs=[_row_spec(nb, F),
                           pl.BlockSpec((1, F), lambda i: (0, 0))],
                out_shape=[jax.ShapeDtypeStruct((n, F), f32),
                           jax.ShapeDtypeStruct((1, F), f32)],
                scratch_shapes=[pltpu.VMEM((1, F), f32),
                                pltpu.VMEM((1, F), f32)],
            )(agg2, agg2, lat_n, *node_w, *glob_w)
    return lat_n[:, :10], glob[:, :10]
